# baseline (speedup 1.0000x reference)
import functools

import jax
import jax.numpy as jnp
from jax import lax
from jax.experimental import pallas as pl
from jax.experimental.pallas import tpu as pltpu
from jax.experimental.pallas import tpu_sc as plsc

C = 26
V = 100000
E = 16
B = 4096
TOTAL = B * C
NUM_CORES = 2
NUM_SUBCORES = 16
NW = NUM_CORES * NUM_SUBCORES
B_PER_W = TOTAL // NW
CHUNK = 128
N_CHUNKS = B_PER_W // CHUNK
LANES = 16


def _gather_kernel(inp_hbm, off_hbm, tab_hbm, out_hbm, inp_v, off_v, idx_v,
                   rows_v, sem):
    wid = lax.axis_index("s") * NUM_CORES + lax.axis_index("c")
    base = wid * B_PER_W

    pltpu.sync_copy(inp_hbm.at[pl.ds(base, B_PER_W)], inp_v)
    pltpu.sync_copy(off_hbm, off_v)

    def add_body(i, carry):
        s = i * LANES
        idx_v[pl.ds(s, LANES)] = inp_v[pl.ds(s, LANES)] + off_v[pl.ds(s, LANES)]
        return carry

    lax.fori_loop(0, B_PER_W // LANES, add_body, 0)

    def fire_body(j, carry):
        s = j * CHUNK
        pltpu.async_copy(
            tab_hbm.at[idx_v.at[pl.ds(s, CHUNK)]],
            rows_v.at[pl.ds(s, CHUNK)],
            sem,
        )
        return carry

    lax.fori_loop(0, N_CHUNKS, fire_body, 0)
    pltpu.make_async_copy(tab_hbm.at[pl.ds(0, B_PER_W)], rows_v, sem).wait()

    pltpu.sync_copy(rows_v, out_hbm.at[pl.ds(base, B_PER_W)])


@jax.jit
def kernel(inp, tables):
    inp_flat = inp.reshape(TOTAL)
    tab_flat = tables.reshape(C * V, E)
    offsets = jnp.tile(jnp.arange(C, dtype=jnp.int32) * V, B_PER_W // C)

    mesh = plsc.VectorSubcoreMesh(core_axis_name="c", subcore_axis_name="s")
    run = functools.partial(
        pl.kernel,
        mesh=mesh,
        compiler_params=pltpu.CompilerParams(use_tc_tiling_on_sc=False),
        out_type=jax.ShapeDtypeStruct((TOTAL, E), jnp.float32),
        scratch_types=[
            pltpu.VMEM((B_PER_W,), jnp.int32),
            pltpu.VMEM((B_PER_W,), jnp.int32),
            pltpu.VMEM((B_PER_W,), jnp.int32),
            pltpu.VMEM((B_PER_W, E), jnp.float32),
            pltpu.SemaphoreType.DMA,
        ],
    )(_gather_kernel)
    out = run(inp_flat, offsets, tab_flat)
    return out.reshape(B, C * E)
